# baseline (speedup 1.0000x reference)
import jax
import jax.numpy as jnp
from jax import lax
from jax.experimental import pallas as pl
from jax.experimental.pallas import tpu as pltpu
from jax.experimental.pallas import tpu_sc as plsc

NN = 10000
NP = 10240
EE = 320000
HH = 128
BB = 100
FGG = 32
ACTT = 290

NC = 2
NS = 16
NW = NC * NS
KC = 128
CH = 80
EP = NW * KC * CH
RPT = NP // NS

CH0 = 32
CH1 = 2 * CH - CH0
CHMAX = max(CH0, CH1)
TOTC = NW * CH
NCH0TOT = NS * CH0

R = 1024
GRID = NP // R


def _sc_mesh():
  return plsc.VectorSubcoreMesh(core_axis_name="c", subcore_axis_name="s")


def _deg_body(dst_hbm, ones_hbm, zeros_hbm, out_hbm, didx_v, ones_v, acc, sem):
  c = lax.axis_index("c")
  s = lax.axis_index("s")
  wid = c * NS + s
  pltpu.sync_copy(dst_hbm.at[pl.ds(wid * CH, CH)], didx_v)
  pltpu.sync_copy(ones_hbm, ones_v)
  pltpu.sync_copy(zeros_hbm, acc.at[pl.ds(s * RPT, RPT)])
  plsc.subcore_barrier()

  def step(ci, _):
    pltpu.sync_copy(ones_v, acc.at[didx_v.at[ci]], add=True)
    return 0

  lax.fori_loop(0, CH, step, 0)
  plsc.subcore_barrier()
  pltpu.sync_copy(acc.at[pl.ds(s * RPT, RPT)],
                  out_hbm.at[c, pl.ds(s * RPT, RPT)])


def _sc_degree(dst_idx, ones_col, zeros_col):
  k = pl.kernel(
      _deg_body,
      out_type=jax.ShapeDtypeStruct((NC, NP, 1), jnp.float32),
      mesh=_sc_mesh(),
      scratch_types=[
          pltpu.VMEM((CH, KC), jnp.int32),
          pltpu.VMEM((KC, 1), jnp.float32),
          pltpu.VMEM_SHARED((NP, 1), jnp.float32),
          pltpu.SemaphoreType.DMA,
      ],
  )
  return k(dst_idx, ones_col, zeros_col)


def _agg_body(y_hbm, src_hbm, dst_hbm, out_hbm,
              sidx_v, didx_v, rows0, acc, sem):
  c = lax.axis_index("c")
  s = lax.axis_index("s")
  base = jnp.where(c == 0, s * CH0, NCH0TOT + s * CH1)
  pltpu.sync_copy(src_hbm.at[pl.ds(base, CHMAX)], sidx_v)
  pltpu.sync_copy(dst_hbm.at[pl.ds(base, CHMAX)], didx_v)

  def zrow(r, _):
    for j in range(HH // 16):
      rows0[r, pl.ds(16 * j, 16)] = jnp.zeros((16,), jnp.float32)
    return 0

  lax.fori_loop(0, KC, zrow, 0)

  def zcopy(z, _):
    pltpu.sync_copy(rows0, acc.at[pl.ds(s * RPT + z * KC, KC)])
    return 0

  lax.fori_loop(0, RPT // KC, zcopy, 0)
  plsc.subcore_barrier()

  def step(ci, _):
    pltpu.sync_copy(y_hbm.at[sidx_v.at[ci]], rows0)
    pltpu.sync_copy(rows0, acc.at[didx_v.at[ci]], add=True)
    return 0

  lax.cond(c == 0,
           lambda: lax.fori_loop(0, CH0, step, 0),
           lambda: lax.fori_loop(0, CH1, step, 0))
  plsc.subcore_barrier()
  pltpu.sync_copy(acc.at[pl.ds(s * RPT, RPT)],
                  out_hbm.at[c, pl.ds(s * RPT, RPT)])


def _sc_aggregate(y, src_idx, dst_idx):
  k = pl.kernel(
      _agg_body,
      out_type=jax.ShapeDtypeStruct((NC, NP, HH), jnp.float32),
      mesh=_sc_mesh(),
      scratch_types=[
          pltpu.VMEM((CHMAX, KC), jnp.int32),
          pltpu.VMEM((CHMAX, KC), jnp.int32),
          pltpu.VMEM((KC, HH), jnp.float32),
          pltpu.VMEM_SHARED((NP, HH), jnp.float32),
          pltpu.SemaphoreType.DMA,
      ],
  )
  return k(y, src_idx, dst_idx)


def _mm0_body(x_ref, w_ref, degp_ref, y_ref, dinv_ref):
  deg = 1.0 + degp_ref[0] + degp_ref[1]
  dinv = lax.rsqrt(deg)
  xw = jnp.dot(x_ref[...], w_ref[...], preferred_element_type=jnp.float32)
  y_ref[...] = xw * dinv
  dinv_ref[...] = dinv


def _tc_mm0(x, w0, degp):
  return pl.pallas_call(
      _mm0_body,
      grid=(GRID,),
      in_specs=[
          pl.BlockSpec((R, HH), lambda i: (i, 0)),
          pl.BlockSpec((HH, HH), lambda i: (0, 0)),
          pl.BlockSpec((NC, R, 1), lambda i: (0, i, 0)),
      ],
      out_specs=[
          pl.BlockSpec((R, HH), lambda i: (i, 0)),
          pl.BlockSpec((R, 1), lambda i: (i, 0)),
      ],
      out_shape=[
          jax.ShapeDtypeStruct((NP, HH), jnp.float32),
          jax.ShapeDtypeStruct((NP, 1), jnp.float32),
      ],
  )(x, w0, degp)


def _post_gcn(p_ref, y_ref, dinv_ref, b_ref, g_ref, be_ref):
  dinv = dinv_ref[...]
  t = (p_ref[0] + p_ref[1] + y_ref[...]) * dinv + b_ref[...]
  m = jnp.mean(t, axis=-1, keepdims=True)
  d = t - m
  v = jnp.mean(d * d, axis=-1, keepdims=True)
  u = d * lax.rsqrt(v + 1e-5) * g_ref[...] + be_ref[...]
  return jnp.maximum(u, 0.0), dinv


def _layer_body(p_ref, y_ref, dinv_ref, b_ref, g_ref, be_ref, wn_ref, out_ref):
  u, dinv = _post_gcn(p_ref, y_ref, dinv_ref, b_ref, g_ref, be_ref)
  out_ref[...] = jnp.dot(u, wn_ref[...], preferred_element_type=jnp.float32) * dinv


def _tc_layer(p, y, dinv, b, g, be, w_next):
  return pl.pallas_call(
      _layer_body,
      grid=(GRID,),
      in_specs=[
          pl.BlockSpec((NC, R, HH), lambda i: (0, i, 0)),
          pl.BlockSpec((R, HH), lambda i: (i, 0)),
          pl.BlockSpec((R, 1), lambda i: (i, 0)),
          pl.BlockSpec((1, HH), lambda i: (0, 0)),
          pl.BlockSpec((1, HH), lambda i: (0, 0)),
          pl.BlockSpec((1, HH), lambda i: (0, 0)),
          pl.BlockSpec((HH, HH), lambda i: (0, 0)),
      ],
      out_specs=pl.BlockSpec((R, HH), lambda i: (i, 0)),
      out_shape=jax.ShapeDtypeStruct((NP, HH), jnp.float32),
  )(p, y, dinv, b, g, be, w_next)


def _head_body(p_ref, y_ref, dinv_ref, b_ref, g_ref, be_ref, batch_ref, gf_ref,
               hw0_ref, hb0_ref, hw1_ref, hb1_ref, hw2_ref, hb2_ref,
               out_ref, pooled_acc, cnt_acc):
  i = pl.program_id(0)
  u, _ = _post_gcn(p_ref, y_ref, dinv_ref, b_ref, g_ref, be_ref)
  bb = batch_ref[...]
  sel = (lax.broadcasted_iota(jnp.int32, (BB, R), 0) == bb).astype(jnp.float32)
  pooled = jnp.dot(sel, u, preferred_element_type=jnp.float32)
  counts = jnp.sum(sel, axis=1, keepdims=True)

  @pl.when(i == 0)
  def _():
    pooled_acc[...] = jnp.zeros_like(pooled_acc)
    cnt_acc[...] = jnp.zeros_like(cnt_acc)

  pooled_acc[...] += pooled
  cnt_acc[...] += counts

  @pl.when(i == GRID - 1)
  def _():
    mean = pooled_acc[...] / jnp.maximum(cnt_acc[...], 1.0)
    hw0 = hw0_ref[...]
    h = (jnp.dot(mean, hw0[:HH, :], preferred_element_type=jnp.float32)
         + jnp.dot(gf_ref[...], hw0[HH:, :], preferred_element_type=jnp.float32))
    h = jnp.maximum(h + hb0_ref[...], 0.0)
    h = jnp.maximum(
        jnp.dot(h, hw1_ref[...], preferred_element_type=jnp.float32)
        + hb1_ref[...], 0.0)
    out_ref[...] = (
        jnp.dot(h, hw2_ref[...], preferred_element_type=jnp.float32)
        + hb2_ref[...])


def _tc_head(p, y, dinv, b, g, be, batch2d, gf, hw0, hb0, hw1, hb1, hw2, hb2):
  def full(shape):
    return pl.BlockSpec(shape, lambda i: tuple(0 for _ in shape))
  return pl.pallas_call(
      _head_body,
      grid=(GRID,),
      in_specs=[
          pl.BlockSpec((NC, R, HH), lambda i: (0, i, 0)),
          pl.BlockSpec((R, HH), lambda i: (i, 0)),
          pl.BlockSpec((R, 1), lambda i: (i, 0)),
          full((1, HH)),
          full((1, HH)),
          full((1, HH)),
          pl.BlockSpec((1, R), lambda i: (0, i)),
          full((BB, FGG)),
          full((HH + FGG, HH)),
          full((1, HH)),
          full((HH, HH)),
          full((1, HH)),
          full((HH, ACTT)),
          full((1, ACTT)),
      ],
      out_specs=pl.BlockSpec((BB, ACTT), lambda i: (0, 0)),
      out_shape=jax.ShapeDtypeStruct((BB, ACTT), jnp.float32),
      scratch_shapes=[
          pltpu.VMEM((BB, HH), jnp.float32),
          pltpu.VMEM((BB, 1), jnp.float32),
      ],
  )(p, y, dinv, b, g, be, batch2d, gf, hw0, hb0, hw1, hb1, hw2, hb2)


def kernel(node_features, edge_index, global_features, batch,
           W0, b0, g0, be0, W1, b1, g1, be1, W2, b2, g2, be2,
           hW0, hb0, hW1, hb1, hW2, hb2):
  f32 = jnp.float32

  xpad = jnp.pad(node_features, ((0, NP - NN), (0, 0)))
  order = jnp.argsort(edge_index[0])
  src = jnp.pad(edge_index[0][order], (0, EP - EE), constant_values=NN)
  dst = jnp.pad(edge_index[1][order], (0, EP - EE), constant_values=NP - 1)
  src = src.reshape(TOTC, KC)
  dst = dst.reshape(TOTC, KC)
  batch2d = jnp.pad(batch, (0, NP - NN), constant_values=BB).reshape(1, NP)

  ones_col = jnp.ones((KC, 1), f32)
  zeros_col = jnp.zeros((RPT, 1), f32)

  def row(a):
    return a.reshape(1, -1)

  degp = _sc_degree(dst, ones_col, zeros_col)
  y, dinv = _tc_mm0(xpad, W0, degp)

  p = _sc_aggregate(y, src, dst)
  y = _tc_layer(p, y, dinv, row(b0), row(g0), row(be0), W1)

  p = _sc_aggregate(y, src, dst)
  y = _tc_layer(p, y, dinv, row(b1), row(g1), row(be1), W2)

  p = _sc_aggregate(y, src, dst)
  out = _tc_head(p, y, dinv, row(b2), row(g2), row(be2),
                 batch2d, global_features,
                 hW0, row(hb0), hW1, row(hb1), hW2, row(hb2))
  return out


# speedup vs baseline: 1.5497x; 1.5497x over previous
import jax
import jax.numpy as jnp
from jax import lax
from jax.experimental import pallas as pl
from jax.experimental.pallas import tpu as pltpu
from jax.experimental.pallas import tpu_sc as plsc

NN = 10000
NP = 10240
EE = 320000
HH = 128
BB = 100
FGG = 32
ACTT = 290

NC = 2
NS = 16
NW = NC * NS
KC = 128
CH = 80
EP = NW * KC * CH
RPT = NP // NS

TOTC = NW * CH

KC2 = 256
CH2 = 40
STG = 8
NSTG = CH2 // STG

R = 1024
GRID = NP // R


def _sc_mesh():
  return plsc.VectorSubcoreMesh(core_axis_name="c", subcore_axis_name="s")


def _deg_body(dst_hbm, ones_hbm, zeros_hbm, out_hbm, didx_v, ones_v, acc, sem):
  c = lax.axis_index("c")
  s = lax.axis_index("s")
  wid = c * NS + s
  pltpu.sync_copy(dst_hbm.at[pl.ds(wid * CH, CH)], didx_v)
  pltpu.sync_copy(ones_hbm, ones_v)
  pltpu.sync_copy(zeros_hbm, acc.at[pl.ds(s * RPT, RPT)])
  plsc.subcore_barrier()

  def step(ci, _):
    pltpu.sync_copy(ones_v, acc.at[didx_v.at[ci]], add=True)
    return 0

  lax.fori_loop(0, CH, step, 0)
  plsc.subcore_barrier()
  pltpu.sync_copy(acc.at[pl.ds(s * RPT, RPT)],
                  out_hbm.at[c, pl.ds(s * RPT, RPT)])


def _sc_degree(dst_idx, ones_col, zeros_col):
  k = pl.kernel(
      _deg_body,
      out_type=jax.ShapeDtypeStruct((NC, NP, 1), jnp.float32),
      mesh=_sc_mesh(),
      scratch_types=[
          pltpu.VMEM((CH, KC), jnp.int32),
          pltpu.VMEM((KC, 1), jnp.float32),
          pltpu.VMEM_SHARED((NP, 1), jnp.float32),
          pltpu.SemaphoreType.DMA,
      ],
  )
  return k(dst_idx, ones_col, zeros_col)


def _agg_body(y_hbm, src_hbm, dst_hbm, out_hbm,
              sidx_v, didx_v, rows0, acc, sem):
  c = lax.axis_index("c")
  s = lax.axis_index("s")
  wid = c * NS + s

  def zrow(r, _):
    for j in range(HH // 16):
      rows0[r, pl.ds(16 * j, 16)] = jnp.zeros((16,), jnp.float32)
    return 0

  lax.fori_loop(0, KC2, zrow, 0)

  def zcopy(z, _):
    pltpu.sync_copy(rows0.at[pl.ds(0, KC)], acc.at[pl.ds(s * RPT + z * KC, KC)])
    return 0

  lax.fori_loop(0, RPT // KC, zcopy, 0)
  plsc.subcore_barrier()

  for stage in range(NSTG):
    off = (wid * CH2 + stage * STG) * KC2
    pltpu.sync_copy(src_hbm.at[pl.ds(off, STG * KC2)], sidx_v)
    pltpu.sync_copy(dst_hbm.at[pl.ds(off, STG * KC2)], didx_v)

    def step(ci, _):
      pltpu.sync_copy(y_hbm.at[sidx_v.at[pl.ds(ci * KC2, KC2)]], rows0)
      pltpu.sync_copy(rows0.at[pl.ds(0, KC)],
                      acc.at[didx_v.at[pl.ds(ci * KC2, KC)]], add=True)
      pltpu.sync_copy(rows0.at[pl.ds(KC, KC)],
                      acc.at[didx_v.at[pl.ds(ci * KC2 + KC, KC)]], add=True)
      return 0

    lax.fori_loop(0, STG, step, 0)

  plsc.subcore_barrier()
  pltpu.sync_copy(acc.at[pl.ds(s * RPT, RPT)],
                  out_hbm.at[c, pl.ds(s * RPT, RPT)])


def _sc_aggregate(y, src_flat, dst_flat):
  k = pl.kernel(
      _agg_body,
      out_type=jax.ShapeDtypeStruct((NC, NP, HH), jnp.float32),
      mesh=_sc_mesh(),
      scratch_types=[
          pltpu.VMEM((STG * KC2,), jnp.int32),
          pltpu.VMEM((STG * KC2,), jnp.int32),
          pltpu.VMEM((KC2, HH), jnp.float32),
          pltpu.VMEM_SHARED((NP, HH), jnp.float32),
          pltpu.SemaphoreType.DMA,
      ],
  )
  return k(y, src_flat, dst_flat)


def _mm0_body(x_ref, w_ref, degp_ref, y_ref, dinv_ref):
  deg = 1.0 + degp_ref[0] + degp_ref[1]
  dinv = lax.rsqrt(deg)
  xw = jnp.dot(x_ref[...], w_ref[...], preferred_element_type=jnp.float32)
  y_ref[...] = xw * dinv
  dinv_ref[...] = dinv


def _tc_mm0(x, w0, degp):
  return pl.pallas_call(
      _mm0_body,
      grid=(GRID,),
      in_specs=[
          pl.BlockSpec((R, HH), lambda i: (i, 0)),
          pl.BlockSpec((HH, HH), lambda i: (0, 0)),
          pl.BlockSpec((NC, R, 1), lambda i: (0, i, 0)),
      ],
      out_specs=[
          pl.BlockSpec((R, HH), lambda i: (i, 0)),
          pl.BlockSpec((R, 1), lambda i: (i, 0)),
      ],
      out_shape=[
          jax.ShapeDtypeStruct((NP, HH), jnp.float32),
          jax.ShapeDtypeStruct((NP, 1), jnp.float32),
      ],
  )(x, w0, degp)


def _post_gcn(p_ref, y_ref, dinv_ref, b_ref, g_ref, be_ref):
  dinv = dinv_ref[...]
  t = (p_ref[0] + p_ref[1] + y_ref[...]) * dinv + b_ref[...]
  m = jnp.mean(t, axis=-1, keepdims=True)
  d = t - m
  v = jnp.mean(d * d, axis=-1, keepdims=True)
  u = d * lax.rsqrt(v + 1e-5) * g_ref[...] + be_ref[...]
  return jnp.maximum(u, 0.0), dinv


def _layer_body(p_ref, y_ref, dinv_ref, b_ref, g_ref, be_ref, wn_ref, out_ref):
  u, dinv = _post_gcn(p_ref, y_ref, dinv_ref, b_ref, g_ref, be_ref)
  out_ref[...] = jnp.dot(u, wn_ref[...], preferred_element_type=jnp.float32) * dinv


def _tc_layer(p, y, dinv, b, g, be, w_next):
  return pl.pallas_call(
      _layer_body,
      grid=(GRID,),
      in_specs=[
          pl.BlockSpec((NC, R, HH), lambda i: (0, i, 0)),
          pl.BlockSpec((R, HH), lambda i: (i, 0)),
          pl.BlockSpec((R, 1), lambda i: (i, 0)),
          pl.BlockSpec((1, HH), lambda i: (0, 0)),
          pl.BlockSpec((1, HH), lambda i: (0, 0)),
          pl.BlockSpec((1, HH), lambda i: (0, 0)),
          pl.BlockSpec((HH, HH), lambda i: (0, 0)),
      ],
      out_specs=pl.BlockSpec((R, HH), lambda i: (i, 0)),
      out_shape=jax.ShapeDtypeStruct((NP, HH), jnp.float32),
  )(p, y, dinv, b, g, be, w_next)


def _head_body(p_ref, y_ref, dinv_ref, b_ref, g_ref, be_ref, batch_ref, gf_ref,
               hw0_ref, hb0_ref, hw1_ref, hb1_ref, hw2_ref, hb2_ref,
               out_ref, pooled_acc, cnt_acc):
  i = pl.program_id(0)
  u, _ = _post_gcn(p_ref, y_ref, dinv_ref, b_ref, g_ref, be_ref)
  bb = batch_ref[...]
  sel = (lax.broadcasted_iota(jnp.int32, (BB, R), 0) == bb).astype(jnp.float32)
  pooled = jnp.dot(sel, u, preferred_element_type=jnp.float32)
  counts = jnp.sum(sel, axis=1, keepdims=True)

  @pl.when(i == 0)
  def _():
    pooled_acc[...] = jnp.zeros_like(pooled_acc)
    cnt_acc[...] = jnp.zeros_like(cnt_acc)

  pooled_acc[...] += pooled
  cnt_acc[...] += counts

  @pl.when(i == GRID - 1)
  def _():
    mean = pooled_acc[...] / jnp.maximum(cnt_acc[...], 1.0)
    hw0 = hw0_ref[...]
    h = (jnp.dot(mean, hw0[:HH, :], preferred_element_type=jnp.float32)
         + jnp.dot(gf_ref[...], hw0[HH:, :], preferred_element_type=jnp.float32))
    h = jnp.maximum(h + hb0_ref[...], 0.0)
    h = jnp.maximum(
        jnp.dot(h, hw1_ref[...], preferred_element_type=jnp.float32)
        + hb1_ref[...], 0.0)
    out_ref[...] = (
        jnp.dot(h, hw2_ref[...], preferred_element_type=jnp.float32)
        + hb2_ref[...])


def _tc_head(p, y, dinv, b, g, be, batch2d, gf, hw0, hb0, hw1, hb1, hw2, hb2):
  def full(shape):
    return pl.BlockSpec(shape, lambda i: tuple(0 for _ in shape))
  return pl.pallas_call(
      _head_body,
      grid=(GRID,),
      in_specs=[
          pl.BlockSpec((NC, R, HH), lambda i: (0, i, 0)),
          pl.BlockSpec((R, HH), lambda i: (i, 0)),
          pl.BlockSpec((R, 1), lambda i: (i, 0)),
          full((1, HH)),
          full((1, HH)),
          full((1, HH)),
          pl.BlockSpec((1, R), lambda i: (0, i)),
          full((BB, FGG)),
          full((HH + FGG, HH)),
          full((1, HH)),
          full((HH, HH)),
          full((1, HH)),
          full((HH, ACTT)),
          full((1, ACTT)),
      ],
      out_specs=pl.BlockSpec((BB, ACTT), lambda i: (0, 0)),
      out_shape=jax.ShapeDtypeStruct((BB, ACTT), jnp.float32),
      scratch_shapes=[
          pltpu.VMEM((BB, HH), jnp.float32),
          pltpu.VMEM((BB, 1), jnp.float32),
      ],
  )(p, y, dinv, b, g, be, batch2d, gf, hw0, hb0, hw1, hb1, hw2, hb2)


def kernel(node_features, edge_index, global_features, batch,
           W0, b0, g0, be0, W1, b1, g1, be1, W2, b2, g2, be2,
           hW0, hb0, hW1, hb1, hW2, hb2):
  f32 = jnp.float32

  xpad = jnp.pad(node_features, ((0, NP - NN), (0, 0)))
  src = jnp.pad(edge_index[0], (0, EP - EE), constant_values=NN)
  dst = jnp.pad(edge_index[1], (0, EP - EE), constant_values=NP - 1)
  dst2d = dst.reshape(TOTC, KC)
  batch2d = jnp.pad(batch, (0, NP - NN), constant_values=BB).reshape(1, NP)

  ones_col = jnp.ones((KC, 1), f32)
  zeros_col = jnp.zeros((RPT, 1), f32)

  def row(a):
    return a.reshape(1, -1)

  degp = _sc_degree(dst2d, ones_col, zeros_col)
  y, dinv = _tc_mm0(xpad, W0, degp)

  p = _sc_aggregate(y, src, dst)
  y = _tc_layer(p, y, dinv, row(b0), row(g0), row(be0), W1)

  p = _sc_aggregate(y, src, dst)
  y = _tc_layer(p, y, dinv, row(b1), row(g1), row(be1), W2)

  p = _sc_aggregate(y, src, dst)
  out = _tc_head(p, y, dinv, row(b2), row(g2), row(be2),
                 batch2d, global_features,
                 hW0, row(hb0), hW1, row(hb1), hW2, row(hb2))
  return out


# speedup vs baseline: 1.6580x; 1.0699x over previous
import jax
import jax.numpy as jnp
from jax import lax
from jax.experimental import pallas as pl
from jax.experimental.pallas import tpu as pltpu
from jax.experimental.pallas import tpu_sc as plsc

NN = 10000
NP = 10240
EE = 320000
HH = 128
BB = 100
FGG = 32
ACTT = 290

NC = 2
NS = 16
NW = NC * NS
KC = 128
CH = 80
EP = NW * KC * CH
RPT = NP // NS

TOTC = NW * CH

STG = 8

R = 1024
GRID = NP // R


def _sc_mesh():
  return plsc.VectorSubcoreMesh(core_axis_name="c", subcore_axis_name="s")


def _deg_body(dst_hbm, ones_hbm, zeros_hbm, out_hbm, didx_v, ones_v, acc, sem):
  c = lax.axis_index("c")
  s = lax.axis_index("s")
  wid = c * NS + s
  pltpu.sync_copy(dst_hbm.at[pl.ds(wid * CH, CH)], didx_v)
  pltpu.sync_copy(ones_hbm, ones_v)
  pltpu.sync_copy(zeros_hbm, acc.at[pl.ds(s * RPT, RPT)])
  plsc.subcore_barrier()

  def step(ci, _):
    pltpu.sync_copy(ones_v, acc.at[didx_v.at[ci]], add=True)
    return 0

  lax.fori_loop(0, CH, step, 0)
  plsc.subcore_barrier()
  pltpu.sync_copy(acc.at[pl.ds(s * RPT, RPT)],
                  out_hbm.at[c, pl.ds(s * RPT, RPT)])


def _sc_degree(dst_idx, ones_col, zeros_col):
  k = pl.kernel(
      _deg_body,
      out_type=jax.ShapeDtypeStruct((NC, NP, 1), jnp.float32),
      mesh=_sc_mesh(),
      scratch_types=[
          pltpu.VMEM((CH, KC), jnp.int32),
          pltpu.VMEM((KC, 1), jnp.float32),
          pltpu.VMEM_SHARED((NP, 1), jnp.float32),
          pltpu.SemaphoreType.DMA,
      ],
  )
  return k(dst_idx, ones_col, zeros_col)


def _agg_body(y_hbm, src_hbm, dst_hbm, out_hbm,
              sidx_a, didx_a, sidx_b, didx_b, rows0, rows1, acc, sem):
  c = lax.axis_index("c")
  s = lax.axis_index("s")
  wid = c * NS + s

  def zrow(r, _):
    for j in range(HH // 16):
      rows0[r, pl.ds(16 * j, 16)] = jnp.zeros((16,), jnp.float32)
    return 0

  lax.fori_loop(0, KC, zrow, 0)

  def zcopy(z, _):
    pltpu.sync_copy(rows0, acc.at[pl.ds(s * RPT + z * KC, KC)])
    return 0

  lax.fori_loop(0, RPT // KC, zcopy, 0)
  plsc.subcore_barrier()

  for st in range(CH // STG):
    si, di = (sidx_a, didx_a) if st % 2 == 0 else (sidx_b, didx_b)
    pltpu.sync_copy(src_hbm.at[pl.ds(wid * CH + st * STG, STG)], si)
    pltpu.sync_copy(dst_hbm.at[pl.ds(wid * CH + st * STG, STG)], di)

    def step2(cj, _, si=si, di=di):
      i0 = 2 * cj
      pltpu.sync_copy(y_hbm.at[si.at[i0]], rows0)
      pltpu.sync_copy(rows0, acc.at[di.at[i0]], add=True)
      pltpu.sync_copy(y_hbm.at[si.at[i0 + 1]], rows1)
      pltpu.sync_copy(rows1, acc.at[di.at[i0 + 1]], add=True)
      return 0

    lax.fori_loop(0, STG // 2, step2, 0)

  plsc.subcore_barrier()
  pltpu.sync_copy(acc.at[pl.ds(s * RPT, RPT)],
                  out_hbm.at[c, pl.ds(s * RPT, RPT)])


def _sc_aggregate(y, src_idx, dst_idx):
  k = pl.kernel(
      _agg_body,
      out_type=jax.ShapeDtypeStruct((NC, NP, HH), jnp.float32),
      mesh=_sc_mesh(),
      scratch_types=[
          pltpu.VMEM((STG, KC), jnp.int32),
          pltpu.VMEM((STG, KC), jnp.int32),
          pltpu.VMEM((STG, KC), jnp.int32),
          pltpu.VMEM((STG, KC), jnp.int32),
          pltpu.VMEM((KC, HH), jnp.float32),
          pltpu.VMEM((KC, HH), jnp.float32),
          pltpu.VMEM_SHARED((NP, HH), jnp.float32),
          pltpu.SemaphoreType.DMA,
      ],
  )
  return k(y, src_idx, dst_idx)


def _mm0_body(x_ref, w_ref, degp_ref, y_ref, dinv_ref):
  deg = 1.0 + degp_ref[0] + degp_ref[1]
  dinv = lax.rsqrt(deg)
  xw = jnp.dot(x_ref[...], w_ref[...], preferred_element_type=jnp.float32)
  y_ref[...] = xw * dinv
  dinv_ref[...] = dinv


def _tc_mm0(x, w0, degp):
  return pl.pallas_call(
      _mm0_body,
      grid=(GRID,),
      in_specs=[
          pl.BlockSpec((R, HH), lambda i: (i, 0)),
          pl.BlockSpec((HH, HH), lambda i: (0, 0)),
          pl.BlockSpec((NC, R, 1), lambda i: (0, i, 0)),
      ],
      out_specs=[
          pl.BlockSpec((R, HH), lambda i: (i, 0)),
          pl.BlockSpec((R, 1), lambda i: (i, 0)),
      ],
      out_shape=[
          jax.ShapeDtypeStruct((NP, HH), jnp.float32),
          jax.ShapeDtypeStruct((NP, 1), jnp.float32),
      ],
  )(x, w0, degp)


def _post_gcn(p_ref, y_ref, dinv_ref, b_ref, g_ref, be_ref):
  dinv = dinv_ref[...]
  t = (p_ref[0] + p_ref[1] + y_ref[...]) * dinv + b_ref[...]
  m = jnp.mean(t, axis=-1, keepdims=True)
  d = t - m
  v = jnp.mean(d * d, axis=-1, keepdims=True)
  u = d * lax.rsqrt(v + 1e-5) * g_ref[...] + be_ref[...]
  return jnp.maximum(u, 0.0), dinv


def _layer_body(p_ref, y_ref, dinv_ref, b_ref, g_ref, be_ref, wn_ref, out_ref):
  u, dinv = _post_gcn(p_ref, y_ref, dinv_ref, b_ref, g_ref, be_ref)
  out_ref[...] = jnp.dot(u, wn_ref[...], preferred_element_type=jnp.float32) * dinv


def _tc_layer(p, y, dinv, b, g, be, w_next):
  return pl.pallas_call(
      _layer_body,
      grid=(GRID,),
      in_specs=[
          pl.BlockSpec((NC, R, HH), lambda i: (0, i, 0)),
          pl.BlockSpec((R, HH), lambda i: (i, 0)),
          pl.BlockSpec((R, 1), lambda i: (i, 0)),
          pl.BlockSpec((1, HH), lambda i: (0, 0)),
          pl.BlockSpec((1, HH), lambda i: (0, 0)),
          pl.BlockSpec((1, HH), lambda i: (0, 0)),
          pl.BlockSpec((HH, HH), lambda i: (0, 0)),
      ],
      out_specs=pl.BlockSpec((R, HH), lambda i: (i, 0)),
      out_shape=jax.ShapeDtypeStruct((NP, HH), jnp.float32),
  )(p, y, dinv, b, g, be, w_next)


def _head_body(p_ref, y_ref, dinv_ref, b_ref, g_ref, be_ref, batch_ref, gf_ref,
               hw0_ref, hb0_ref, hw1_ref, hb1_ref, hw2_ref, hb2_ref,
               out_ref, pooled_acc, cnt_acc):
  i = pl.program_id(0)
  u, _ = _post_gcn(p_ref, y_ref, dinv_ref, b_ref, g_ref, be_ref)
  bb = batch_ref[...]
  sel = (lax.broadcasted_iota(jnp.int32, (BB, R), 0) == bb).astype(jnp.float32)
  pooled = jnp.dot(sel, u, preferred_element_type=jnp.float32)
  counts = jnp.sum(sel, axis=1, keepdims=True)

  @pl.when(i == 0)
  def _():
    pooled_acc[...] = jnp.zeros_like(pooled_acc)
    cnt_acc[...] = jnp.zeros_like(cnt_acc)

  pooled_acc[...] += pooled
  cnt_acc[...] += counts

  @pl.when(i == GRID - 1)
  def _():
    mean = pooled_acc[...] / jnp.maximum(cnt_acc[...], 1.0)
    hw0 = hw0_ref[...]
    h = (jnp.dot(mean, hw0[:HH, :], preferred_element_type=jnp.float32)
         + jnp.dot(gf_ref[...], hw0[HH:, :], preferred_element_type=jnp.float32))
    h = jnp.maximum(h + hb0_ref[...], 0.0)
    h = jnp.maximum(
        jnp.dot(h, hw1_ref[...], preferred_element_type=jnp.float32)
        + hb1_ref[...], 0.0)
    out_ref[...] = (
        jnp.dot(h, hw2_ref[...], preferred_element_type=jnp.float32)
        + hb2_ref[...])


def _tc_head(p, y, dinv, b, g, be, batch2d, gf, hw0, hb0, hw1, hb1, hw2, hb2):
  def full(shape):
    return pl.BlockSpec(shape, lambda i: tuple(0 for _ in shape))
  return pl.pallas_call(
      _head_body,
      grid=(GRID,),
      in_specs=[
          pl.BlockSpec((NC, R, HH), lambda i: (0, i, 0)),
          pl.BlockSpec((R, HH), lambda i: (i, 0)),
          pl.BlockSpec((R, 1), lambda i: (i, 0)),
          full((1, HH)),
          full((1, HH)),
          full((1, HH)),
          pl.BlockSpec((1, R), lambda i: (0, i)),
          full((BB, FGG)),
          full((HH + FGG, HH)),
          full((1, HH)),
          full((HH, HH)),
          full((1, HH)),
          full((HH, ACTT)),
          full((1, ACTT)),
      ],
      out_specs=pl.BlockSpec((BB, ACTT), lambda i: (0, 0)),
      out_shape=jax.ShapeDtypeStruct((BB, ACTT), jnp.float32),
      scratch_shapes=[
          pltpu.VMEM((BB, HH), jnp.float32),
          pltpu.VMEM((BB, 1), jnp.float32),
      ],
  )(p, y, dinv, b, g, be, batch2d, gf, hw0, hb0, hw1, hb1, hw2, hb2)


def kernel(node_features, edge_index, global_features, batch,
           W0, b0, g0, be0, W1, b1, g1, be1, W2, b2, g2, be2,
           hW0, hb0, hW1, hb1, hW2, hb2):
  f32 = jnp.float32

  xpad = jnp.pad(node_features, ((0, NP - NN), (0, 0)))
  src = jnp.pad(edge_index[0], (0, EP - EE), constant_values=NN)
  dst = jnp.pad(edge_index[1], (0, EP - EE), constant_values=NP - 1)
  src = src.reshape(TOTC, KC)
  dst = dst.reshape(TOTC, KC)
  batch2d = jnp.pad(batch, (0, NP - NN), constant_values=BB).reshape(1, NP)

  ones_col = jnp.ones((KC, 1), f32)
  zeros_col = jnp.zeros((RPT, 1), f32)

  def row(a):
    return a.reshape(1, -1)

  degp = _sc_degree(dst, ones_col, zeros_col)
  y, dinv = _tc_mm0(xpad, W0, degp)

  p = _sc_aggregate(y, src, dst)
  y = _tc_layer(p, y, dinv, row(b0), row(g0), row(be0), W1)

  p = _sc_aggregate(y, src, dst)
  y = _tc_layer(p, y, dinv, row(b1), row(g1), row(be1), W2)

  p = _sc_aggregate(y, src, dst)
  out = _tc_head(p, y, dinv, row(b2), row(g2), row(be2),
                 batch2d, global_features,
                 hW0, row(hb0), hW1, row(hb1), hW2, row(hb2))
  return out
